# baseline (device time: 44350 ns/iter reference)
import jax
import jax.numpy as jnp
from jax import lax
from jax.experimental import pallas as pl
from jax.experimental.pallas import tpu as pltpu

B, S, H, D = 2, 512, 8, 64
BH = B * H
SCALE = D ** -0.5
CHUNK = 32
HCHUNK = (S // 2) // CHUNK
HROWS = S // 2


def kernel(Q, K, V):
    Qs = jnp.transpose((Q * SCALE).astype(jnp.bfloat16), (0, 2, 1, 3)).reshape(BH, S, D)
    Kb = jnp.transpose(K.astype(jnp.bfloat16), (0, 2, 1, 3)).reshape(BH, S, D)
    Vb = jnp.transpose(V.astype(jnp.bfloat16), (0, 2, 1, 3)).reshape(BH, S, D)
    Vaug = jnp.concatenate([Vb, jnp.ones((BH, S, 1), jnp.bfloat16)], axis=2)

    def body(q_ref, k_ref, vaug_ref, out_ref,
             krecv, vrecv, o_acc,
             ysend_sems, xsend_sems, yrecv_sems, xrecv_sems):
        my_x = lax.axis_index("x")
        my_y = lax.axis_index("y")
        my_z = lax.axis_index("z")
        y_partner = (my_x, 1 - my_y, my_z)
        x_partner = (1 - my_x, my_y, my_z)

        barrier_sem = pltpu.get_barrier_semaphore()
        for nbr in (y_partner, x_partner):
            pl.semaphore_signal(
                barrier_sem, inc=1,
                device_id=nbr, device_id_type=pl.DeviceIdType.MESH,
            )
        pl.semaphore_wait(barrier_sem, 2)

        myrow = my_x * HROWS
        otrow = (1 - my_x) * HROWS

        def chunk_rdma(src, dst, row, send_sem, recv_sem, target):
            return pltpu.make_async_remote_copy(
                src_ref=src.at[:, pl.ds(row, CHUNK), :],
                dst_ref=dst.at[:, pl.ds(row, CHUNK), :],
                send_sem=send_sem,
                recv_sem=recv_sem,
                device_id=target,
                device_id_type=pl.DeviceIdType.MESH,
            )

        y_rdmas = []
        for jj in range(HCHUNK):
            row = myrow + jj * CHUNK
            rk = chunk_rdma(k_ref, krecv, row,
                            ysend_sems.at[2 * jj], yrecv_sems.at[2 * jj],
                            y_partner)
            rv = chunk_rdma(vaug_ref, vrecv, row,
                            ysend_sems.at[2 * jj + 1], yrecv_sems.at[2 * jj + 1],
                            y_partner)
            rk.start()
            rv.start()
            y_rdmas.append((rk, rv))

        o_acc[...] = jnp.zeros((BH, S, D + 1), jnp.float32)

        def attend(i, k, vaug, final=False):
            s = lax.dot_general(
                q_ref[i], k, (((1,), (1,)), ((), ())),
                preferred_element_type=jnp.float32,
            )
            p = jnp.exp(s).astype(jnp.bfloat16)
            o = lax.dot_general(
                p, vaug, (((1,), (0,)), ((), ())),
                preferred_element_type=jnp.float32,
            )
            if final:
                acc = o_acc[i] + o
                out_ref[i] = acc[:, :D] / acc[:, D:D + 1]
            else:
                o_acc[i] = o_acc[i] + o

        x_rdmas = []
        for jj in range(HCHUNK):
            row = myrow + jj * CHUNK
            y_rdmas[jj][0].wait_recv()
            y_rdmas[jj][1].wait_recv()
            rk = chunk_rdma(krecv, krecv, row,
                            xsend_sems.at[2 * jj], xrecv_sems.at[2 * jj],
                            x_partner)
            rv = chunk_rdma(vrecv, vrecv, row,
                            xsend_sems.at[2 * jj + 1], xrecv_sems.at[2 * jj + 1],
                            x_partner)
            rk.start()
            rv.start()
            x_rdmas.append((rk, rv))
            attend(2 * jj, k_ref[2 * jj], vaug_ref[2 * jj])
            attend(2 * jj + 1, k_ref[2 * jj + 1], vaug_ref[2 * jj + 1])

        for i in range(BH):
            attend(i,
                   krecv[i, pl.ds(myrow, HROWS), :],
                   vrecv[i, pl.ds(myrow, HROWS), :])

        for jj in range(HCHUNK):
            row = otrow + jj * CHUNK
            chunk_rdma(krecv, krecv, row,
                       xsend_sems.at[2 * jj], xrecv_sems.at[2 * jj],
                       x_partner).wait_recv()
            chunk_rdma(vrecv, vrecv, row,
                       xsend_sems.at[2 * jj + 1], xrecv_sems.at[2 * jj + 1],
                       x_partner).wait_recv()

        for i in range(BH):
            attend(i,
                   krecv[i, pl.ds(otrow, HROWS), :],
                   vrecv[i, pl.ds(otrow, HROWS), :],
                   final=True)

        for jj in range(HCHUNK):
            y_rdmas[jj][0].wait_send()
            y_rdmas[jj][1].wait_send()
            x_rdmas[jj][0].wait_send()
            x_rdmas[jj][1].wait_send()

    out = pl.pallas_call(
        body,
        out_shape=jax.ShapeDtypeStruct((BH, S, D), jnp.float32),
        in_specs=[pl.BlockSpec(memory_space=pltpu.VMEM)] * 3,
        out_specs=pl.BlockSpec(memory_space=pltpu.VMEM),
        scratch_shapes=[
            pltpu.VMEM((BH, S, D), jnp.bfloat16),
            pltpu.VMEM((BH, S, D + 1), jnp.bfloat16),
            pltpu.VMEM((BH, S, D + 1), jnp.float32),
            pltpu.SemaphoreType.DMA((2 * HCHUNK,)),
            pltpu.SemaphoreType.DMA((2 * HCHUNK,)),
            pltpu.SemaphoreType.DMA((2 * HCHUNK,)),
            pltpu.SemaphoreType.DMA((2 * HCHUNK,)),
        ],
        compiler_params=pltpu.CompilerParams(collective_id=0),
    )(Qs, Kb, Vaug)

    return jnp.transpose(out.reshape(B, H, S, D), (0, 2, 1, 3))


# device time: 44313 ns/iter; 1.0008x vs baseline; 1.0008x over previous
import jax
import jax.numpy as jnp
from jax import lax
from jax.experimental import pallas as pl
from jax.experimental.pallas import tpu as pltpu

B, S, H, D = 2, 512, 8, 64
BH = B * H
SCALE = D ** -0.5
CHUNK = 32
HCHUNK = (S // 2) // CHUNK
HROWS = S // 2


def kernel(Q, K, V):
    Qs = jnp.transpose((Q * SCALE).astype(jnp.bfloat16), (0, 2, 1, 3)).reshape(BH, S, D)
    Kb = jnp.transpose(K.astype(jnp.bfloat16), (0, 2, 1, 3)).reshape(BH, S, D)
    Vb = jnp.transpose(V.astype(jnp.bfloat16), (0, 2, 1, 3)).reshape(BH, S, D)
    Vaug = jnp.concatenate([Vb, jnp.ones((BH, S, 1), jnp.bfloat16)], axis=2)

    def body(q_ref, k_ref, vaug_ref, out_ref,
             krecv, vrecv, o_acc,
             ysend_sems, xsend_sems, yrecv_sems, xrecv_sems):
        my_x = lax.axis_index("x")
        my_y = lax.axis_index("y")
        my_z = lax.axis_index("z")
        y_partner = (my_x, 1 - my_y, my_z)
        x_partner = (1 - my_x, my_y, my_z)

        barrier_sem = pltpu.get_barrier_semaphore()
        for nbr in (y_partner, x_partner):
            pl.semaphore_signal(
                barrier_sem, inc=1,
                device_id=nbr, device_id_type=pl.DeviceIdType.MESH,
            )
        pl.semaphore_wait(barrier_sem, 2)

        myrow = my_x * HROWS
        otrow = (1 - my_x) * HROWS

        def chunk_rdma(src, dst, row, send_sem, recv_sem, target):
            return pltpu.make_async_remote_copy(
                src_ref=src.at[:, pl.ds(row, CHUNK), :],
                dst_ref=dst.at[:, pl.ds(row, CHUNK), :],
                send_sem=send_sem,
                recv_sem=recv_sem,
                device_id=target,
                device_id_type=pl.DeviceIdType.MESH,
            )

        y_rdmas = []
        for jj in range(HCHUNK):
            row = myrow + jj * CHUNK
            rk = chunk_rdma(k_ref, krecv, row,
                            ysend_sems.at[2 * jj], yrecv_sems.at[2 * jj],
                            y_partner)
            rv = chunk_rdma(vaug_ref, vrecv, row,
                            ysend_sems.at[2 * jj + 1], yrecv_sems.at[2 * jj + 1],
                            y_partner)
            rk.start()
            rv.start()
            y_rdmas.append((rk, rv))

        def attend(i, k, vaug, first=False, final=False):
            s = lax.dot_general(
                q_ref[i], k, (((1,), (1,)), ((), ())),
                preferred_element_type=jnp.float32,
            )
            p = jnp.exp(s).astype(jnp.bfloat16)
            o = lax.dot_general(
                p, vaug, (((1,), (0,)), ((), ())),
                preferred_element_type=jnp.float32,
            )
            if final:
                acc = o_acc[i] + o
                out_ref[i] = acc[:, :D] / acc[:, D:D + 1]
            elif first:
                o_acc[i] = o
            else:
                o_acc[i] = o_acc[i] + o

        x_rdmas = []
        for jj in range(HCHUNK):
            row = myrow + jj * CHUNK
            y_rdmas[jj][0].wait_recv()
            y_rdmas[jj][1].wait_recv()
            rk = chunk_rdma(krecv, krecv, row,
                            xsend_sems.at[2 * jj], xrecv_sems.at[2 * jj],
                            x_partner)
            rv = chunk_rdma(vrecv, vrecv, row,
                            xsend_sems.at[2 * jj + 1], xrecv_sems.at[2 * jj + 1],
                            x_partner)
            rk.start()
            rv.start()
            x_rdmas.append((rk, rv))
            attend(2 * jj, k_ref[2 * jj], vaug_ref[2 * jj], first=True)
            attend(2 * jj + 1, k_ref[2 * jj + 1], vaug_ref[2 * jj + 1], first=True)

        for i in range(BH):
            attend(i,
                   krecv[i, pl.ds(myrow, HROWS), :],
                   vrecv[i, pl.ds(myrow, HROWS), :])

        for jj in range(HCHUNK):
            row = otrow + jj * CHUNK
            chunk_rdma(krecv, krecv, row,
                       xsend_sems.at[2 * jj], xrecv_sems.at[2 * jj],
                       x_partner).wait_recv()
            chunk_rdma(vrecv, vrecv, row,
                       xsend_sems.at[2 * jj + 1], xrecv_sems.at[2 * jj + 1],
                       x_partner).wait_recv()

        for i in range(BH):
            attend(i,
                   krecv[i, pl.ds(otrow, HROWS), :],
                   vrecv[i, pl.ds(otrow, HROWS), :],
                   final=True)

        for jj in range(HCHUNK):
            y_rdmas[jj][0].wait_send()
            y_rdmas[jj][1].wait_send()
            x_rdmas[jj][0].wait_send()
            x_rdmas[jj][1].wait_send()

    out = pl.pallas_call(
        body,
        out_shape=jax.ShapeDtypeStruct((BH, S, D), jnp.float32),
        in_specs=[pl.BlockSpec(memory_space=pltpu.VMEM)] * 3,
        out_specs=pl.BlockSpec(memory_space=pltpu.VMEM),
        scratch_shapes=[
            pltpu.VMEM((BH, S, D), jnp.bfloat16),
            pltpu.VMEM((BH, S, D + 1), jnp.bfloat16),
            pltpu.VMEM((BH, S, D + 1), jnp.float32),
            pltpu.SemaphoreType.DMA((2 * HCHUNK,)),
            pltpu.SemaphoreType.DMA((2 * HCHUNK,)),
            pltpu.SemaphoreType.DMA((2 * HCHUNK,)),
            pltpu.SemaphoreType.DMA((2 * HCHUNK,)),
        ],
        compiler_params=pltpu.CompilerParams(collective_id=0),
    )(Qs, Kb, Vaug)

    return jnp.transpose(out.reshape(B, H, S, D), (0, 2, 1, 3))
